# Initial kernel scaffold; baseline (speedup 1.0000x reference)
import jax
import jax.numpy as jnp
from jax.experimental import pallas as pl


def kernel(x, edge_index, graph_ids, bn_gamma, bn_beta, W1_0, b1_0, W2_0, b2_0, W1_1, b1_1, W2_1, b2_1, W1_2, b1_2, W2_2, b2_2, centroids, fc_W, fc_b):
    raise NotImplementedError("write your pallas kernel here")


# speedup vs baseline: 3.3768x; 3.3768x over previous
import jax
import jax.numpy as jnp
from jax import lax
from jax.experimental import pallas as pl
from jax.experimental.pallas import tpu as pltpu
from jax.experimental.pallas import tpu_sc as plsc

N = 10000
E = 320000
D = 128
G = 64
K = 100

NC, NS, L = 2, 16, 16
NW = NC * NS
EPW = E // NW
CH = 80
NCHK = EPW // CH
NPAD = 10240
RPS = NPAD // NS

f32 = jnp.float32


def _sc_mesh():
    return plsc.VectorSubcoreMesh(core_axis_name="c", subcore_axis_name="s",
                                  num_cores=NC, num_subcores=NS)


_SC_PARAMS = pltpu.CompilerParams(use_tc_tiling_on_sc=False)


def _prep_body(x_ref, g_ref, bt_ref, W1_ref, b1_ref, A_ref, B_ref):
    x = x_ref[...]
    mean = jnp.mean(x, axis=0, keepdims=True)
    xc = x - mean
    var = jnp.mean(xc * xc, axis=0, keepdims=True)
    h = xc * (g_ref[...] * lax.rsqrt(var + 1e-5)) + bt_ref[...]
    cin = D
    Wa = W1_ref[0:cin, :]
    Wb = W1_ref[cin:2 * cin, :]
    A_ref[...] = jnp.dot(h, Wa - Wb, preferred_element_type=f32) + b1_ref[...]
    B_ref[...] = jnp.dot(h, Wb, preferred_element_type=f32)


def _tc_prep(x, g, bt, W1, b1, c1):
    return pl.pallas_call(
        _prep_body,
        out_shape=(jax.ShapeDtypeStruct((N, c1), f32),
                   jax.ShapeDtypeStruct((N, c1), f32)),
    )(x, g, bt, W1, b1)


def _mlp_body(t_ref, W2_ref, b2_ref, u_ref):
    u_ref[...] = jnp.maximum(
        jnp.dot(t_ref[...], W2_ref[...], preferred_element_type=f32)
        + b2_ref[...], 0.0)


def _tc_mlp(t, W2, b2, c1, c2, blk=8000):
    nblk = E // blk
    return pl.pallas_call(
        _mlp_body,
        grid=(nblk,),
        in_specs=[pl.BlockSpec((blk, c1), lambda i: (i, 0)),
                  pl.BlockSpec((c1, c2), lambda i: (0, 0)),
                  pl.BlockSpec((1, c2), lambda i: (0, 0))],
        out_specs=pl.BlockSpec((blk, c2), lambda i: (i, 0)),
        out_shape=jax.ShapeDtypeStruct((E, c2), f32),
    )(t, W2, b2)


def _combine_body(agg_ref, deg_ref, W1_ref, b1_ref, A_ref, B_ref):
    deg = deg_ref[0, 0:N, 0:1] + deg_ref[1, 0:N, 0:1]
    h = (agg_ref[0, 0:N, :] + agg_ref[1, 0:N, :]) / jnp.maximum(deg, 1.0)
    cin = h.shape[1]
    Wa = W1_ref[0:cin, :]
    Wb = W1_ref[cin:2 * cin, :]
    A_ref[...] = jnp.dot(h, Wa - Wb, preferred_element_type=f32) + b1_ref[...]
    B_ref[...] = jnp.dot(h, Wb, preferred_element_type=f32)


def _tc_combine(agg, deg, W1, b1, c1n):
    return pl.pallas_call(
        _combine_body,
        out_shape=(jax.ShapeDtypeStruct((N, c1n), f32),
                   jax.ShapeDtypeStruct((N, c1n), f32)),
    )(agg, deg, W1, b1)


def _final_body(agg_ref, deg_ref, gid_ref, c_ref, fw_ref, fb_ref, o_ref):
    deg = deg_ref[0, 0:N, 0:1] + deg_ref[1, 0:N, 0:1]
    h = (agg_ref[0, 0:N, :] + agg_ref[1, 0:N, :]) / jnp.maximum(deg, 1.0)
    c = c_ref[...]
    x2 = jnp.sum(h * h, axis=1, keepdims=True)
    c2 = jnp.sum(c * c, axis=1, keepdims=True).T
    hc = lax.dot_general(h, c, (((1,), (1,)), ((), ())),
                         preferred_element_type=f32)
    dists = jnp.sqrt(jnp.maximum(x2 + c2 - 2.0 * hc, 1e-12))
    onehot = (gid_ref[...] ==
              lax.broadcasted_iota(jnp.int32, (G, N), 0)).astype(f32)
    gsum = jnp.dot(onehot, dists, preferred_element_type=f32)
    gcnt = jnp.sum(onehot, axis=1, keepdims=True)
    pooled = gsum / jnp.maximum(gcnt, 1.0)
    o_ref[...] = jnp.dot(pooled, fw_ref[...],
                         preferred_element_type=f32) + fb_ref[...]


def _tc_final(agg, deg, gid, cent, fw, fb):
    return pl.pallas_call(
        _final_body,
        out_shape=jax.ShapeDtypeStruct((G, 2), f32),
    )(agg, deg, gid, cent, fw, fb)


def _gather_body(c1, A_hbm, B_hbm, dst_hbm, src_hbm, out_hbm,
                 idxd_v, idxs_v, bufA, bufB, sem):
    cid = lax.axis_index("c")
    sid = lax.axis_index("s")
    wid = cid * NS + sid
    pltpu.sync_copy(dst_hbm.at[wid], idxd_v)
    pltpu.sync_copy(src_hbm.at[wid], idxs_v)

    def chunk(j, carry):
        base = wid * EPW + j * CH
        pltpu.async_copy(A_hbm.at[idxd_v.at[j]], bufA, sem).wait()
        pltpu.async_copy(B_hbm.at[idxs_v.at[j]], bufB, sem).wait()
        for r in range(CH):
            for k in range(c1 // L):
                s = pl.ds(k * L, L)
                bufA[r, s] = jnp.maximum(bufA[r, s] + bufB[r, s], 0.0)
        pltpu.sync_copy(bufA, out_hbm.at[pl.ds(base, CH)])
        return carry

    lax.fori_loop(0, NCHK, chunk, 0)


def _sc_gather(A, B, dst2, src2, c1):
    def body(*refs):
        _gather_body(c1, *refs)
    return pl.kernel(
        body,
        out_type=jax.ShapeDtypeStruct((E, c1), f32),
        mesh=_sc_mesh(),
        compiler_params=_SC_PARAMS,
        scratch_types=[
            pltpu.VMEM((NCHK, CH), jnp.int32),
            pltpu.VMEM((NCHK, CH), jnp.int32),
            pltpu.VMEM((CH, c1), f32),
            pltpu.VMEM((CH, c1), f32),
            pltpu.SemaphoreType.DMA,
        ],
    )(A, B, dst2, src2)


def _scatter_body(c2, with_deg, u_hbm, dst_hbm, agg_out, deg_out,
                  idx_v, ubuf, zbuf, zdeg, ones_v, agg_sp, deg_sp):
    cid = lax.axis_index("c")
    sid = lax.axis_index("s")
    wid = cid * NS + sid
    z = jnp.zeros((L,), f32)
    for r in range(RPS // 5):
        for k in range(c2 // L):
            zbuf[r, pl.ds(k * L, L)] = z
    for j in range(5):
        pltpu.sync_copy(zbuf, agg_sp.at[pl.ds(sid * RPS + j * 128, 128)])
    if with_deg:
        one = jnp.full((L,), 1.0, f32)
        for r in range(RPS // 5):
            zdeg[r, :] = z
        for r in range(CH):
            ones_v[r, :] = one
        for j in range(5):
            pltpu.sync_copy(zdeg, deg_sp.at[pl.ds(sid * RPS + j * 128, 128)])
    pltpu.sync_copy(dst_hbm.at[wid], idx_v)
    plsc.subcore_barrier()

    def chunk(j, carry):
        base = wid * EPW + j * CH
        pltpu.sync_copy(u_hbm.at[pl.ds(base, CH)], ubuf)
        pltpu.sync_copy(ubuf, agg_sp.at[idx_v.at[j]], add=True)
        if with_deg:
            pltpu.sync_copy(ones_v, deg_sp.at[idx_v.at[j]], add=True)
        return carry

    lax.fori_loop(0, NCHK, chunk, 0)
    plsc.subcore_barrier()
    sl = pl.ds(sid * RPS, RPS)
    pltpu.sync_copy(agg_sp.at[sl], agg_out.at[cid, sl])
    if with_deg:
        pltpu.sync_copy(deg_sp.at[sl], deg_out.at[cid, sl])


def _sc_scatter(u, dst2, c2, with_deg):
    out_type = [jax.ShapeDtypeStruct((NC, NPAD, c2), f32)]
    scratch = [
        pltpu.VMEM((NCHK, CH), jnp.int32),
        pltpu.VMEM((CH, c2), f32),
        pltpu.VMEM((RPS // 5, c2), f32),
        pltpu.VMEM((RPS // 5, L), f32),
        pltpu.VMEM((CH, L), f32),
        pltpu.VMEM_SHARED((NPAD, c2), f32),
        pltpu.VMEM_SHARED((NPAD, L), f32),
    ]
    if with_deg:
        out_type.append(jax.ShapeDtypeStruct((NC, NPAD, L), f32))

        def body(u_hbm, dst_hbm, agg_out, deg_out, *s):
            _scatter_body(c2, True, u_hbm, dst_hbm, agg_out, deg_out, *s)
    else:
        def body(u_hbm, dst_hbm, agg_out, *s):
            _scatter_body(c2, False, u_hbm, dst_hbm, agg_out, None, *s)
    return pl.kernel(
        body,
        out_type=tuple(out_type),
        mesh=_sc_mesh(),
        compiler_params=_SC_PARAMS,
        scratch_types=scratch,
    )(u, dst2)


def kernel(x, edge_index, graph_ids, bn_gamma, bn_beta,
           W1_0, b1_0, W2_0, b2_0,
           W1_1, b1_1, W2_1, b2_1,
           W1_2, b1_2, W2_2, b2_2,
           centroids, fc_W, fc_b):
    ei = edge_index.astype(jnp.int32)
    src2 = ei[0].reshape(NW, NCHK, CH)
    dst2 = ei[1].reshape(NW, NCHK, CH)
    gid = graph_ids.astype(jnp.int32).reshape(1, N)
    g2 = bn_gamma.reshape(1, D)
    bt2 = bn_beta.reshape(1, D)

    A0, B0 = _tc_prep(x, g2, bt2, W1_0, b1_0.reshape(1, -1), 32)
    t0 = _sc_gather(A0, B0, dst2, src2, 32)
    u0 = _tc_mlp(t0, W2_0, b2_0.reshape(1, -1), 32, 32)
    agg0, deg = _sc_scatter(u0, dst2, 32, True)

    A1, B1 = _tc_combine(agg0, deg, W1_1, b1_1.reshape(1, -1), 32)
    t1 = _sc_gather(A1, B1, dst2, src2, 32)
    u1 = _tc_mlp(t1, W2_1, b2_1.reshape(1, -1), 32, 32)
    (agg1,) = _sc_scatter(u1, dst2, 32, False)

    A2, B2 = _tc_combine(agg1, deg, W1_2, b1_2.reshape(1, -1), 64)
    t2 = _sc_gather(A2, B2, dst2, src2, 64)
    u2 = _tc_mlp(t2, W2_2, b2_2.reshape(1, -1), 64, 64)
    (agg2,) = _sc_scatter(u2, dst2, 64, False)

    return _tc_final(agg2, deg, gid, centroids, fc_W, fc_b.reshape(1, 2))


# speedup vs baseline: 4.8484x; 1.4358x over previous
import jax
import jax.numpy as jnp
from jax import lax
from jax.experimental import pallas as pl
from jax.experimental.pallas import tpu as pltpu
from jax.experimental.pallas import tpu_sc as plsc

N = 10000
E = 320000
D = 128
G = 64
K = 100

NC, NS, L = 2, 16, 16
NW = NC * NS
EPW = E // NW
CH = 80
NCHK = EPW // CH
NPAD = 10240
RPS = NPAD // NS
MAC = 5
NMAC = NCHK // MAC

f32 = jnp.float32


def _sc_mesh():
    return plsc.VectorSubcoreMesh(core_axis_name="c", subcore_axis_name="s",
                                  num_cores=NC, num_subcores=NS)


_SC_PARAMS = pltpu.CompilerParams(use_tc_tiling_on_sc=False)


def _prep_body(x_ref, g_ref, bt_ref, W1_ref, b1_ref, A_ref, B_ref):
    x = x_ref[...]
    mean = jnp.mean(x, axis=0, keepdims=True)
    xc = x - mean
    var = jnp.mean(xc * xc, axis=0, keepdims=True)
    h = xc * (g_ref[...] * lax.rsqrt(var + 1e-5)) + bt_ref[...]
    cin = D
    Wa = W1_ref[0:cin, :]
    Wb = W1_ref[cin:2 * cin, :]
    A_ref[...] = jnp.dot(h, Wa - Wb, preferred_element_type=f32) + b1_ref[...]
    B_ref[...] = jnp.dot(h, Wb, preferred_element_type=f32)


def _tc_prep(x, g, bt, W1, b1, c1):
    return pl.pallas_call(
        _prep_body,
        out_shape=(jax.ShapeDtypeStruct((N, c1), f32),
                   jax.ShapeDtypeStruct((N, c1), f32)),
    )(x, g, bt, W1, b1)


def _mlp_body(t_ref, W2_ref, b2_ref, u_ref):
    u_ref[...] = jnp.maximum(
        jnp.dot(t_ref[...], W2_ref[...], preferred_element_type=f32)
        + b2_ref[...], 0.0)


def _tc_mlp(t, W2, b2, c1, c2, blk=8000):
    nblk = E // blk
    return pl.pallas_call(
        _mlp_body,
        grid=(nblk,),
        in_specs=[pl.BlockSpec((blk, c1), lambda i: (i, 0)),
                  pl.BlockSpec((c1, c2), lambda i: (0, 0)),
                  pl.BlockSpec((1, c2), lambda i: (0, 0))],
        out_specs=pl.BlockSpec((blk, c2), lambda i: (i, 0)),
        out_shape=jax.ShapeDtypeStruct((E, c2), f32),
    )(t, W2, b2)


def _combine_body(agg_ref, deg_ref, W1_ref, b1_ref, A_ref, B_ref):
    deg = deg_ref[0, 0:N, 0:1] + deg_ref[1, 0:N, 0:1]
    h = (agg_ref[0, 0:N, :] + agg_ref[1, 0:N, :]) / jnp.maximum(deg, 1.0)
    cin = h.shape[1]
    Wa = W1_ref[0:cin, :]
    Wb = W1_ref[cin:2 * cin, :]
    A_ref[...] = jnp.dot(h, Wa - Wb, preferred_element_type=f32) + b1_ref[...]
    B_ref[...] = jnp.dot(h, Wb, preferred_element_type=f32)


def _tc_combine(agg, deg, W1, b1, c1n):
    return pl.pallas_call(
        _combine_body,
        out_shape=(jax.ShapeDtypeStruct((N, c1n), f32),
                   jax.ShapeDtypeStruct((N, c1n), f32)),
    )(agg, deg, W1, b1)


def _final_body(agg_ref, deg_ref, gid_ref, c_ref, fw_ref, fb_ref, o_ref):
    deg = deg_ref[0, 0:N, 0:1] + deg_ref[1, 0:N, 0:1]
    h = (agg_ref[0, 0:N, :] + agg_ref[1, 0:N, :]) / jnp.maximum(deg, 1.0)
    c = c_ref[...]
    x2 = jnp.sum(h * h, axis=1, keepdims=True)
    c2 = jnp.sum(c * c, axis=1, keepdims=True).T
    hc = lax.dot_general(h, c, (((1,), (1,)), ((), ())),
                         preferred_element_type=f32)
    dists = jnp.sqrt(jnp.maximum(x2 + c2 - 2.0 * hc, 1e-12))
    onehot = (gid_ref[...] ==
              lax.broadcasted_iota(jnp.int32, (G, N), 0)).astype(f32)
    gsum = jnp.dot(onehot, dists, preferred_element_type=f32)
    gcnt = jnp.sum(onehot, axis=1, keepdims=True)
    pooled = gsum / jnp.maximum(gcnt, 1.0)
    o_ref[...] = jnp.dot(pooled, fw_ref[...],
                         preferred_element_type=f32) + fb_ref[...]


def _tc_final(agg, deg, gid, cent, fw, fb):
    return pl.pallas_call(
        _final_body,
        out_shape=jax.ShapeDtypeStruct((G, 2), f32),
    )(agg, deg, gid, cent, fw, fb)


def _gather_body(c1, A_hbm, B_hbm, dst_hbm, src_hbm, out_hbm,
                 idxd_v, idxs_v, bufA, bufB, sem0, sem1):
    cid = lax.axis_index("c")
    sid = lax.axis_index("s")
    wid = cid * NS + sid
    pltpu.sync_copy(dst_hbm.at[wid], idxd_v)
    pltpu.sync_copy(src_hbm.at[wid], idxs_v)
    sems = (sem0, sem1)

    def fire(j, s):
        pltpu.async_copy(A_hbm.at[idxd_v.at[j]], bufA.at[s], sems[s])
        pltpu.async_copy(B_hbm.at[idxs_v.at[j]], bufB.at[s], sems[s])

    def proc(j, s):
        pltpu.make_async_copy(A_hbm.at[idxd_v.at[j]], bufA.at[s], sems[s]).wait()
        pltpu.make_async_copy(B_hbm.at[idxs_v.at[j]], bufB.at[s], sems[s]).wait()
        for r in range(CH):
            for k in range(c1 // L):
                d = pl.ds(k * L, L)
                bufA[s, r, d] = jnp.maximum(bufA[s, r, d] + bufB[s, r, d], 0.0)
        pltpu.sync_copy(bufA.at[s], out_hbm.at[pl.ds(wid * EPW + j * CH, CH)])

    fire(0, 0)
    fire(1, 1)

    def body(j2, carry):
        j = 2 * j2
        proc(j, 0)

        @pl.when(j + 2 < NCHK)
        def _():
            fire(j + 2, 0)

        @pl.when(j + 1 < NCHK)
        def _():
            proc(j + 1, 1)

        @pl.when(j + 3 < NCHK)
        def _():
            fire(j + 3, 1)

        return carry

    lax.fori_loop(0, (NCHK + 1) // 2, body, 0)


def _sc_gather(A, B, dst2, src2, c1):
    def body(*refs):
        _gather_body(c1, *refs)
    return pl.kernel(
        body,
        out_type=jax.ShapeDtypeStruct((E, c1), f32),
        mesh=_sc_mesh(),
        compiler_params=_SC_PARAMS,
        scratch_types=[
            pltpu.VMEM((NCHK, CH), jnp.int32),
            pltpu.VMEM((NCHK, CH), jnp.int32),
            pltpu.VMEM((2, CH, c1), f32),
            pltpu.VMEM((2, CH, c1), f32),
            pltpu.SemaphoreType.DMA,
            pltpu.SemaphoreType.DMA,
        ],
    )(A, B, dst2, src2)


def _scatter_body(c2, with_deg, u_hbm, dst_hbm, agg_out, deg_out,
                  idx_v, ubuf, zbuf, zdeg, ones_v, agg_sp, deg_sp,
                  rsem0, rsem1, asem0, asem1):
    cid = lax.axis_index("c")
    sid = lax.axis_index("s")
    wid = cid * NS + sid
    z = jnp.zeros((L,), f32)
    for r in range(RPS // 5):
        for k in range(c2 // L):
            zbuf[r, pl.ds(k * L, L)] = z
    for j in range(5):
        pltpu.sync_copy(zbuf, agg_sp.at[pl.ds(sid * RPS + j * 128, 128)])
    if with_deg:
        one = jnp.full((L,), 1.0, f32)
        for r in range(RPS // 5):
            zdeg[r, :] = z
        for r in range(CH):
            ones_v[r, :] = one
        for j in range(5):
            pltpu.sync_copy(zdeg, deg_sp.at[pl.ds(sid * RPS + j * 128, 128)])
    pltpu.sync_copy(dst_hbm.at[wid], idx_v)
    plsc.subcore_barrier()
    rsems = (rsem0, rsem1)
    asems = (asem0, asem1)

    def fire_read(m, s):
        pltpu.async_copy(
            u_hbm.at[pl.ds(wid * EPW + m * MAC * CH, MAC * CH)],
            ubuf.at[s], rsems[s])

    def proc(m, s):
        pltpu.make_async_copy(
            u_hbm.at[pl.ds(wid * EPW + m * MAC * CH, MAC * CH)],
            ubuf.at[s], rsems[s]).wait()
        for k in range(MAC):
            pltpu.async_copy(ubuf.at[s, pl.ds(k * CH, CH)],
                             agg_sp.at[idx_v.at[m * MAC + k]], asems[s],
                             add=True)
            if with_deg:
                pltpu.async_copy(ones_v, deg_sp.at[idx_v.at[m * MAC + k]],
                                 asems[s], add=True)
        for k in range(MAC):
            pltpu.make_async_copy(ubuf.at[s, pl.ds(k * CH, CH)],
                                  agg_sp.at[idx_v.at[m * MAC + k]],
                                  asems[s]).wait()
            if with_deg:
                pltpu.make_async_copy(ones_v, deg_sp.at[idx_v.at[m * MAC + k]],
                                      asems[s]).wait()

        @pl.when(m + 2 < NMAC)
        def _():
            fire_read(m + 2, s)

    fire_read(0, 0)
    fire_read(1, 1)

    def body(j2, carry):
        m = 2 * j2
        proc(m, 0)

        @pl.when(m + 1 < NMAC)
        def _():
            proc(m + 1, 1)

        return carry

    lax.fori_loop(0, (NMAC + 1) // 2, body, 0)
    plsc.subcore_barrier()
    sl = pl.ds(sid * RPS, RPS)
    pltpu.sync_copy(agg_sp.at[sl], agg_out.at[cid, sl])
    if with_deg:
        pltpu.sync_copy(deg_sp.at[sl], deg_out.at[cid, sl])


def _sc_scatter(u, dst2, c2, with_deg):
    out_type = [jax.ShapeDtypeStruct((NC, NPAD, c2), f32)]
    scratch = [
        pltpu.VMEM((NCHK, CH), jnp.int32),
        pltpu.VMEM((2, MAC * CH, c2), f32),
        pltpu.VMEM((RPS // 5, c2), f32),
        pltpu.VMEM((RPS // 5, L), f32),
        pltpu.VMEM((CH, L), f32),
        pltpu.VMEM_SHARED((NPAD, c2), f32),
        pltpu.VMEM_SHARED((NPAD, L), f32),
        pltpu.SemaphoreType.DMA,
        pltpu.SemaphoreType.DMA,
        pltpu.SemaphoreType.DMA,
        pltpu.SemaphoreType.DMA,
    ]
    if with_deg:
        out_type.append(jax.ShapeDtypeStruct((NC, NPAD, L), f32))

        def body(u_hbm, dst_hbm, agg_out, deg_out, *s):
            _scatter_body(c2, True, u_hbm, dst_hbm, agg_out, deg_out, *s)
    else:
        def body(u_hbm, dst_hbm, agg_out, *s):
            _scatter_body(c2, False, u_hbm, dst_hbm, agg_out, None, *s)
    return pl.kernel(
        body,
        out_type=tuple(out_type),
        mesh=_sc_mesh(),
        compiler_params=_SC_PARAMS,
        scratch_types=scratch,
    )(u, dst2)


def kernel(x, edge_index, graph_ids, bn_gamma, bn_beta,
           W1_0, b1_0, W2_0, b2_0,
           W1_1, b1_1, W2_1, b2_1,
           W1_2, b1_2, W2_2, b2_2,
           centroids, fc_W, fc_b):
    ei = edge_index.astype(jnp.int32)
    src2 = ei[0].reshape(NW, NCHK, CH)
    dst2 = ei[1].reshape(NW, NCHK, CH)
    gid = graph_ids.astype(jnp.int32).reshape(1, N)
    g2 = bn_gamma.reshape(1, D)
    bt2 = bn_beta.reshape(1, D)

    A0, B0 = _tc_prep(x, g2, bt2, W1_0, b1_0.reshape(1, -1), 32)
    t0 = _sc_gather(A0, B0, dst2, src2, 32)
    u0 = _tc_mlp(t0, W2_0, b2_0.reshape(1, -1), 32, 32)
    agg0, deg = _sc_scatter(u0, dst2, 32, True)

    A1, B1 = _tc_combine(agg0, deg, W1_1, b1_1.reshape(1, -1), 32)
    t1 = _sc_gather(A1, B1, dst2, src2, 32)
    u1 = _tc_mlp(t1, W2_1, b2_1.reshape(1, -1), 32, 32)
    (agg1,) = _sc_scatter(u1, dst2, 32, False)

    A2, B2 = _tc_combine(agg1, deg, W1_2, b1_2.reshape(1, -1), 64)
    t2 = _sc_gather(A2, B2, dst2, src2, 64)
    u2 = _tc_mlp(t2, W2_2, b2_2.reshape(1, -1), 64, 64)
    (agg2,) = _sc_scatter(u2, dst2, 64, False)

    return _tc_final(agg2, deg, gid, centroids, fc_W, fc_b.reshape(1, 2))


# speedup vs baseline: 4.9265x; 1.0161x over previous
import jax
import jax.numpy as jnp
from jax import lax
from jax.experimental import pallas as pl
from jax.experimental.pallas import tpu as pltpu
from jax.experimental.pallas import tpu_sc as plsc

N = 10000
E = 320000
D = 128
G = 64
K = 100

NC, NS, L = 2, 16, 16
NW = NC * NS
EPW = E // NW
CH = 80
NCHK = EPW // CH
NPAD = 10240
RPS = NPAD // NS
MAC = 5
NMAC = NCHK // MAC

f32 = jnp.float32


def _sc_mesh():
    return plsc.VectorSubcoreMesh(core_axis_name="c", subcore_axis_name="s",
                                  num_cores=NC, num_subcores=NS)


_SC_PARAMS = pltpu.CompilerParams(use_tc_tiling_on_sc=False)


def _prep_body(x_ref, g_ref, bt_ref, W1_ref, b1_ref, A_ref, B_ref):
    x = x_ref[...]
    mean = jnp.mean(x, axis=0, keepdims=True)
    xc = x - mean
    var = jnp.mean(xc * xc, axis=0, keepdims=True)
    h = xc * (g_ref[...] * lax.rsqrt(var + 1e-5)) + bt_ref[...]
    cin = D
    Wa = W1_ref[0:cin, :]
    Wb = W1_ref[cin:2 * cin, :]
    A_ref[...] = jnp.dot(h, Wa - Wb, preferred_element_type=f32) + b1_ref[...]
    B_ref[...] = jnp.dot(h, Wb, preferred_element_type=f32)


def _tc_prep(x, g, bt, W1, b1, c1):
    return pl.pallas_call(
        _prep_body,
        out_shape=(jax.ShapeDtypeStruct((N, c1), f32),
                   jax.ShapeDtypeStruct((N, c1), f32)),
    )(x, g, bt, W1, b1)


def _mlp_body(t_ref, W2_ref, b2_ref, u_ref):
    u_ref[...] = jnp.maximum(
        jnp.dot(t_ref[...], W2_ref[...], preferred_element_type=f32)
        + b2_ref[...], 0.0)


def _tc_mlp(t, W2, b2, c1, c2, blk=8000):
    nblk = E // blk
    return pl.pallas_call(
        _mlp_body,
        grid=(nblk,),
        in_specs=[pl.BlockSpec((blk, c1), lambda i: (i, 0)),
                  pl.BlockSpec((c1, c2), lambda i: (0, 0)),
                  pl.BlockSpec((1, c2), lambda i: (0, 0))],
        out_specs=pl.BlockSpec((blk, c2), lambda i: (i, 0)),
        out_shape=jax.ShapeDtypeStruct((E, c2), f32),
    )(t, W2, b2)


def _combine_body(agg_ref, deg_ref, W1_ref, b1_ref, A_ref, B_ref):
    deg = deg_ref[0, 0:N, 0:1] + deg_ref[1, 0:N, 0:1]
    h = (agg_ref[0, 0:N, :] + agg_ref[1, 0:N, :]) / jnp.maximum(deg, 1.0)
    cin = h.shape[1]
    Wa = W1_ref[0:cin, :]
    Wb = W1_ref[cin:2 * cin, :]
    A_ref[...] = jnp.dot(h, Wa - Wb, preferred_element_type=f32) + b1_ref[...]
    B_ref[...] = jnp.dot(h, Wb, preferred_element_type=f32)


def _tc_combine(agg, deg, W1, b1, c1n):
    return pl.pallas_call(
        _combine_body,
        out_shape=(jax.ShapeDtypeStruct((N, c1n), f32),
                   jax.ShapeDtypeStruct((N, c1n), f32)),
    )(agg, deg, W1, b1)


def _final_body(agg_ref, deg_ref, gid_ref, c_ref, fw_ref, fb_ref, o_ref):
    deg = deg_ref[0, 0:N, 0:1] + deg_ref[1, 0:N, 0:1]
    h = (agg_ref[0, 0:N, :] + agg_ref[1, 0:N, :]) / jnp.maximum(deg, 1.0)
    c = c_ref[...]
    x2 = jnp.sum(h * h, axis=1, keepdims=True)
    c2 = jnp.sum(c * c, axis=1, keepdims=True).T
    hc = lax.dot_general(h, c, (((1,), (1,)), ((), ())),
                         preferred_element_type=f32)
    dists = jnp.sqrt(jnp.maximum(x2 + c2 - 2.0 * hc, 1e-12))
    onehot = (gid_ref[...] ==
              lax.broadcasted_iota(jnp.int32, (G, N), 0)).astype(f32)
    gsum = jnp.dot(onehot, dists, preferred_element_type=f32)
    gcnt = jnp.sum(onehot, axis=1, keepdims=True)
    pooled = gsum / jnp.maximum(gcnt, 1.0)
    o_ref[...] = jnp.dot(pooled, fw_ref[...],
                         preferred_element_type=f32) + fb_ref[...]


def _tc_final(agg, deg, gid, cent, fw, fb):
    return pl.pallas_call(
        _final_body,
        out_shape=jax.ShapeDtypeStruct((G, 2), f32),
    )(agg, deg, gid, cent, fw, fb)


def _gather_body(c1, A_hbm, B_hbm, dst_hbm, src_hbm, out_hbm,
                 idxd_v, idxs_v, bufA, bufB, bufO, sem0, sem1, ssem0, ssem1):
    cid = lax.axis_index("c")
    sid = lax.axis_index("s")
    wid = cid * NS + sid
    pltpu.sync_copy(dst_hbm.at[wid], idxd_v)
    pltpu.sync_copy(src_hbm.at[wid], idxs_v)
    sems = (sem0, sem1)
    ssems = (ssem0, ssem1)

    def out_slice(j):
        return out_hbm.at[pl.ds(wid * EPW + j * CH, CH)]

    def fire(j, s):
        pltpu.async_copy(A_hbm.at[idxd_v.at[j]], bufA.at[s], sems[s])
        pltpu.async_copy(B_hbm.at[idxs_v.at[j]], bufB.at[s], sems[s])

    def proc(j, s):
        pltpu.make_async_copy(A_hbm.at[idxd_v.at[j]], bufA.at[s], sems[s]).wait()
        pltpu.make_async_copy(B_hbm.at[idxs_v.at[j]], bufB.at[s], sems[s]).wait()

        @pl.when(j >= 2)
        def _():
            pltpu.make_async_copy(bufO.at[s], out_slice(j - 2), ssems[s]).wait()

        for r in range(CH):
            for k in range(c1 // L):
                d = pl.ds(k * L, L)
                bufO[s, r, d] = jnp.maximum(bufA[s, r, d] + bufB[s, r, d], 0.0)
        pltpu.async_copy(bufO.at[s], out_slice(j), ssems[s])

    fire(0, 0)
    fire(1, 1)

    def body(j2, carry):
        j = 2 * j2
        proc(j, 0)

        @pl.when(j + 2 < NCHK)
        def _():
            fire(j + 2, 0)

        @pl.when(j + 1 < NCHK)
        def _():
            proc(j + 1, 1)

        @pl.when(j + 3 < NCHK)
        def _():
            fire(j + 3, 1)

        return carry

    lax.fori_loop(0, (NCHK + 1) // 2, body, 0)
    pltpu.make_async_copy(bufO.at[0], out_slice(NCHK - 1), ssem0).wait()
    pltpu.make_async_copy(bufO.at[1], out_slice(NCHK - 2), ssem1).wait()


def _sc_gather(A, B, dst2, src2, c1):
    def body(*refs):
        _gather_body(c1, *refs)
    return pl.kernel(
        body,
        out_type=jax.ShapeDtypeStruct((E, c1), f32),
        mesh=_sc_mesh(),
        compiler_params=_SC_PARAMS,
        scratch_types=[
            pltpu.VMEM((NCHK, CH), jnp.int32),
            pltpu.VMEM((NCHK, CH), jnp.int32),
            pltpu.VMEM((2, CH, c1), f32),
            pltpu.VMEM((2, CH, c1), f32),
            pltpu.VMEM((2, CH, c1), f32),
            pltpu.SemaphoreType.DMA,
            pltpu.SemaphoreType.DMA,
            pltpu.SemaphoreType.DMA,
            pltpu.SemaphoreType.DMA,
        ],
    )(A, B, dst2, src2)


def _scatter_body(c2, with_deg, u_hbm, dst_hbm, agg_out, deg_out,
                  idx_v, ubuf, zbuf, zdeg, ones_v, agg_sp, deg_sp,
                  rsem0, rsem1, asem0, asem1):
    cid = lax.axis_index("c")
    sid = lax.axis_index("s")
    wid = cid * NS + sid
    z = jnp.zeros((L,), f32)
    for r in range(RPS // 5):
        for k in range(c2 // L):
            zbuf[r, pl.ds(k * L, L)] = z
    for j in range(5):
        pltpu.sync_copy(zbuf, agg_sp.at[pl.ds(sid * RPS + j * 128, 128)])
    if with_deg:
        one = jnp.full((L,), 1.0, f32)
        for r in range(RPS // 5):
            zdeg[r, :] = z
        for r in range(CH):
            ones_v[r, :] = one
        for j in range(5):
            pltpu.sync_copy(zdeg, deg_sp.at[pl.ds(sid * RPS + j * 128, 128)])
    pltpu.sync_copy(dst_hbm.at[wid], idx_v)
    plsc.subcore_barrier()
    rsems = (rsem0, rsem1)
    asems = (asem0, asem1)

    def fire_read(m, s):
        pltpu.async_copy(
            u_hbm.at[pl.ds(wid * EPW + m * MAC * CH, MAC * CH)],
            ubuf.at[s], rsems[s])

    def proc(m, s):
        pltpu.make_async_copy(
            u_hbm.at[pl.ds(wid * EPW + m * MAC * CH, MAC * CH)],
            ubuf.at[s], rsems[s]).wait()
        for k in range(MAC):
            pltpu.async_copy(ubuf.at[s, pl.ds(k * CH, CH)],
                             agg_sp.at[idx_v.at[m * MAC + k]], asems[s],
                             add=True)
            if with_deg:
                pltpu.async_copy(ones_v, deg_sp.at[idx_v.at[m * MAC + k]],
                                 asems[s], add=True)
        for k in range(MAC):
            pltpu.make_async_copy(ubuf.at[s, pl.ds(k * CH, CH)],
                                  agg_sp.at[idx_v.at[m * MAC + k]],
                                  asems[s]).wait()
            if with_deg:
                pltpu.make_async_copy(ones_v, deg_sp.at[idx_v.at[m * MAC + k]],
                                      asems[s]).wait()

        @pl.when(m + 2 < NMAC)
        def _():
            fire_read(m + 2, s)

    fire_read(0, 0)
    fire_read(1, 1)

    def body(j2, carry):
        m = 2 * j2
        proc(m, 0)

        @pl.when(m + 1 < NMAC)
        def _():
            proc(m + 1, 1)

        return carry

    lax.fori_loop(0, (NMAC + 1) // 2, body, 0)
    plsc.subcore_barrier()
    sl = pl.ds(sid * RPS, RPS)
    pltpu.sync_copy(agg_sp.at[sl], agg_out.at[cid, sl])
    if with_deg:
        pltpu.sync_copy(deg_sp.at[sl], deg_out.at[cid, sl])


def _sc_scatter(u, dst2, c2, with_deg):
    out_type = [jax.ShapeDtypeStruct((NC, NPAD, c2), f32)]
    scratch = [
        pltpu.VMEM((NCHK, CH), jnp.int32),
        pltpu.VMEM((2, MAC * CH, c2), f32),
        pltpu.VMEM((RPS // 5, c2), f32),
        pltpu.VMEM((RPS // 5, L), f32),
        pltpu.VMEM((CH, L), f32),
        pltpu.VMEM_SHARED((NPAD, c2), f32),
        pltpu.VMEM_SHARED((NPAD, L), f32),
        pltpu.SemaphoreType.DMA,
        pltpu.SemaphoreType.DMA,
        pltpu.SemaphoreType.DMA,
        pltpu.SemaphoreType.DMA,
    ]
    if with_deg:
        out_type.append(jax.ShapeDtypeStruct((NC, NPAD, L), f32))

        def body(u_hbm, dst_hbm, agg_out, deg_out, *s):
            _scatter_body(c2, True, u_hbm, dst_hbm, agg_out, deg_out, *s)
    else:
        def body(u_hbm, dst_hbm, agg_out, *s):
            _scatter_body(c2, False, u_hbm, dst_hbm, agg_out, None, *s)
    return pl.kernel(
        body,
        out_type=tuple(out_type),
        mesh=_sc_mesh(),
        compiler_params=_SC_PARAMS,
        scratch_types=scratch,
    )(u, dst2)


def kernel(x, edge_index, graph_ids, bn_gamma, bn_beta,
           W1_0, b1_0, W2_0, b2_0,
           W1_1, b1_1, W2_1, b2_1,
           W1_2, b1_2, W2_2, b2_2,
           centroids, fc_W, fc_b):
    ei = edge_index.astype(jnp.int32)
    src2 = ei[0].reshape(NW, NCHK, CH)
    dst2 = ei[1].reshape(NW, NCHK, CH)
    gid = graph_ids.astype(jnp.int32).reshape(1, N)
    g2 = bn_gamma.reshape(1, D)
    bt2 = bn_beta.reshape(1, D)

    A0, B0 = _tc_prep(x, g2, bt2, W1_0, b1_0.reshape(1, -1), 32)
    t0 = _sc_gather(A0, B0, dst2, src2, 32)
    u0 = _tc_mlp(t0, W2_0, b2_0.reshape(1, -1), 32, 32)
    agg0, deg = _sc_scatter(u0, dst2, 32, True)

    A1, B1 = _tc_combine(agg0, deg, W1_1, b1_1.reshape(1, -1), 32)
    t1 = _sc_gather(A1, B1, dst2, src2, 32)
    u1 = _tc_mlp(t1, W2_1, b2_1.reshape(1, -1), 32, 32)
    (agg1,) = _sc_scatter(u1, dst2, 32, False)

    A2, B2 = _tc_combine(agg1, deg, W1_2, b1_2.reshape(1, -1), 64)
    t2 = _sc_gather(A2, B2, dst2, src2, 64)
    u2 = _tc_mlp(t2, W2_2, b2_2.reshape(1, -1), 64, 64)
    (agg2,) = _sc_scatter(u2, dst2, 64, False)

    return _tc_final(agg2, deg, gid, centroids, fc_W, fc_b.reshape(1, 2))
